# initial kernel scaffold (unmeasured)
import jax
import jax.numpy as jnp
from jax import lax
from jax.experimental import pallas as pl
from jax.experimental.pallas import tpu as pltpu


def kernel(
    x,
):
    def body(*refs):
        pass

    out_shape = jax.ShapeDtypeStruct(..., jnp.float32)
    return pl.pallas_call(body, out_shape=out_shape)(...)


# baseline (device time: 800916 ns/iter reference)
import jax
import jax.numpy as jnp
from jax import lax
from jax.experimental import pallas as pl
from jax.experimental.pallas import tpu as pltpu

N_DEV = 32
H_R = 16
H_L = 15


def kernel(x):
    m_per, n = x.shape

    def body(x_ref, out_ref, send_r, recv_r, send_l, recv_l, local_sem):
        me = lax.axis_index("i")
        right = lax.rem(me + 1, N_DEV)
        left = lax.rem(me + N_DEV - 1, N_DEV)

        barrier = pltpu.get_barrier_semaphore()
        for nbr in (left, right):
            pl.semaphore_signal(
                barrier, inc=1,
                device_id=(nbr,), device_id_type=pl.DeviceIdType.MESH,
            )
        pl.semaphore_wait(barrier, 2)

        own = pltpu.make_async_copy(
            x_ref, out_ref.at[pl.ds(me * m_per, m_per), :], local_sem
        )
        own.start()
        own.wait()

        def make_rdma(chunk, sems, h, target):
            sl = (pl.ds(chunk * m_per, m_per), slice(None))
            send_sems, recv_sems = sems
            return pltpu.make_async_remote_copy(
                src_ref=out_ref.at[sl],
                dst_ref=out_ref.at[sl],
                send_sem=send_sems.at[h],
                recv_sem=recv_sems.at[h],
                device_id=(target,),
                device_id_type=pl.DeviceIdType.MESH,
            )

        for h in range(H_R):
            rdma_r = make_rdma(
                lax.rem(me - h + N_DEV, N_DEV), (send_r, recv_r), h, right
            )
            rdma_r.start()
            if h < H_L:
                rdma_l = make_rdma(
                    lax.rem(me + h, N_DEV), (send_l, recv_l), h, left
                )
                rdma_l.start()
                rdma_l.wait()
            rdma_r.wait()

    return pl.pallas_call(
        body,
        out_shape=jax.ShapeDtypeStruct((N_DEV * m_per, n), x.dtype),
        in_specs=[pl.BlockSpec(memory_space=pltpu.VMEM)],
        out_specs=pl.BlockSpec(memory_space=pl.ANY),
        scratch_shapes=[
            pltpu.SemaphoreType.DMA((H_R,)),
            pltpu.SemaphoreType.DMA((H_R,)),
            pltpu.SemaphoreType.DMA((H_L,)),
            pltpu.SemaphoreType.DMA((H_L,)),
            pltpu.SemaphoreType.DMA,
        ],
        compiler_params=pltpu.CompilerParams(collective_id=0),
    )(x)


# device time: 448232 ns/iter; 1.7868x vs baseline; 1.7868x over previous
import jax
import jax.numpy as jnp
from jax import lax
from jax.experimental import pallas as pl
from jax.experimental.pallas import tpu as pltpu

N_DEV = 32
H_R = 16
H_L = 15

_Q = {(0, 0): 0, (1, 0): 1, (1, 1): 2, (0, 1): 3,
      (0, 2): 4, (1, 2): 5, (1, 3): 6, (0, 3): 7}


def _logical_id(x: int, y: int, z: int) -> int:
    return z * 8 + _Q[(x, y)]


_P0 = [(y, 0) for y in range(4)] + [(y, 1) for y in reversed(range(4))] + \
      [(y, 2) for y in range(4)] + [(y, 3) for y in reversed(range(4))]
_COORD_CYCLE = [(0, y, z) for (y, z) in _P0] + \
               [(1, y, z) for (y, z) in reversed(_P0)]
for _a, _b in zip(_COORD_CYCLE, _COORD_CYCLE[1:] + _COORD_CYCLE[:1]):
    assert sum(abs(i - j) for i, j in zip(_a, _b)) == 1, (_a, _b)

RING = [_logical_id(*c) for c in _COORD_CYCLE]
POS = [RING.index(i) for i in range(N_DEV)]
assert sorted(RING) == list(range(N_DEV))


def kernel(x):
    m_per, n = x.shape

    def body(x_ref, pos_ref, ring_ref, out_ref,
             send_r, recv_r, send_l, recv_l, local_sem):
        me = lax.axis_index("i")
        p = pos_ref[me]
        right = ring_ref[lax.rem(p + 1, N_DEV)]
        left = ring_ref[lax.rem(p + N_DEV - 1, N_DEV)]

        barrier = pltpu.get_barrier_semaphore()
        for nbr in (left, right):
            pl.semaphore_signal(
                barrier, inc=1,
                device_id=(nbr,), device_id_type=pl.DeviceIdType.MESH,
            )
        pl.semaphore_wait(barrier, 2)

        own = pltpu.make_async_copy(
            x_ref, out_ref.at[pl.ds(me * m_per, m_per), :], local_sem
        )
        own.start()
        own.wait()

        def make_rdma(chunk, sems, h, target):
            sl = (pl.ds(chunk * m_per, m_per), slice(None))
            send_sems, recv_sems = sems
            return pltpu.make_async_remote_copy(
                src_ref=out_ref.at[sl],
                dst_ref=out_ref.at[sl],
                send_sem=send_sems.at[h],
                recv_sem=recv_sems.at[h],
                device_id=(target,),
                device_id_type=pl.DeviceIdType.MESH,
            )

        for h in range(H_R):
            chunk_r = ring_ref[lax.rem(p - h + N_DEV, N_DEV)]
            rdma_r = make_rdma(chunk_r, (send_r, recv_r), h, right)
            rdma_r.start()
            if h < H_L:
                chunk_l = ring_ref[lax.rem(p + h, N_DEV)]
                rdma_l = make_rdma(chunk_l, (send_l, recv_l), h, left)
                rdma_l.start()
                rdma_l.wait()
            rdma_r.wait()

    pos_tab = jnp.asarray(POS, dtype=jnp.int32)
    ring_tab = jnp.asarray(RING, dtype=jnp.int32)

    return pl.pallas_call(
        body,
        out_shape=jax.ShapeDtypeStruct((N_DEV * m_per, n), x.dtype),
        in_specs=[
            pl.BlockSpec(memory_space=pltpu.VMEM),
            pl.BlockSpec(memory_space=pltpu.SMEM),
            pl.BlockSpec(memory_space=pltpu.SMEM),
        ],
        out_specs=pl.BlockSpec(memory_space=pl.ANY),
        scratch_shapes=[
            pltpu.SemaphoreType.DMA((H_R,)),
            pltpu.SemaphoreType.DMA((H_R,)),
            pltpu.SemaphoreType.DMA((H_L,)),
            pltpu.SemaphoreType.DMA((H_L,)),
            pltpu.SemaphoreType.DMA,
        ],
        compiler_params=pltpu.CompilerParams(collective_id=0),
    )(x, pos_tab, ring_tab)


# device time: 413298 ns/iter; 1.9379x vs baseline; 1.0845x over previous
import jax
import jax.numpy as jnp
from jax import lax
from jax.experimental import pallas as pl
from jax.experimental.pallas import tpu as pltpu

N_DEV = 32
H_R = 16
H_L = 15
SUB_R = 2 * H_R
SUB_L = 2 * H_L

_Q = {(0, 0): 0, (1, 0): 1, (1, 1): 2, (0, 1): 3,
      (0, 2): 4, (1, 2): 5, (1, 3): 6, (0, 3): 7}


def _logical_id(x: int, y: int, z: int) -> int:
    return z * 8 + _Q[(x, y)]


_P0 = [(y, 0) for y in range(4)] + [(y, 1) for y in reversed(range(4))] + \
      [(y, 2) for y in range(4)] + [(y, 3) for y in reversed(range(4))]
_COORD_CYCLE = [(0, y, z) for (y, z) in _P0] + \
               [(1, y, z) for (y, z) in reversed(_P0)]
for _a, _b in zip(_COORD_CYCLE, _COORD_CYCLE[1:] + _COORD_CYCLE[:1]):
    assert sum(abs(i - j) for i, j in zip(_a, _b)) == 1, (_a, _b)

RING = [_logical_id(*c) for c in _COORD_CYCLE]
POS = [RING.index(i) for i in range(N_DEV)]
assert sorted(RING) == list(range(N_DEV))


def kernel(x):
    m_per, n = x.shape
    m_sub = m_per // 2

    def body(x_ref, pos_ref, ring_ref, out_ref,
             send_r, recv_r, send_l, recv_l, local_sem):
        me = lax.axis_index("i")
        p = pos_ref[me]
        right = ring_ref[lax.rem(p + 1, N_DEV)]
        left = ring_ref[lax.rem(p + N_DEV - 1, N_DEV)]

        barrier = pltpu.get_barrier_semaphore()
        for nbr in (left, right):
            pl.semaphore_signal(
                barrier, inc=1,
                device_id=(nbr,), device_id_type=pl.DeviceIdType.MESH,
            )
        pl.semaphore_wait(barrier, 2)

        own = pltpu.make_async_copy(
            x_ref, out_ref.at[pl.ds(me * m_per, m_per), :], local_sem
        )
        own.start()

        def sub_slice(chunk, half):
            return (pl.ds(chunk * m_per + half * m_sub, m_sub), slice(None))

        def send_desc(t, rightward):
            k, half = t // 2, t % 2
            if rightward:
                chunk = ring_ref[lax.rem(p - k + N_DEV, N_DEV)]
                sems, tgt = (send_r, recv_r), right
            else:
                chunk = ring_ref[lax.rem(p + k, N_DEV)]
                sems, tgt = (send_l, recv_l), left
            dst = out_ref.at[sub_slice(chunk, half)]
            if t < 2:
                src = x_ref.at[pl.ds(half * m_sub, m_sub), :]
            else:
                src = out_ref.at[sub_slice(chunk, half)]
            return pltpu.make_async_remote_copy(
                src_ref=src, dst_ref=dst,
                send_sem=sems[0].at[t], recv_sem=sems[1].at[t],
                device_id=(tgt,), device_id_type=pl.DeviceIdType.MESH,
            )

        def recv_desc(t, rightward):
            k, half = t // 2, t % 2
            if rightward:
                chunk = ring_ref[lax.rem(p - 1 - k + N_DEV, N_DEV)]
                sems, frm = (send_r, recv_r), left
            else:
                chunk = ring_ref[lax.rem(p + 1 + k, N_DEV)]
                sems, frm = (send_l, recv_l), right
            sl = out_ref.at[sub_slice(chunk, half)]
            return pltpu.make_async_remote_copy(
                src_ref=sl, dst_ref=sl,
                send_sem=sems[0].at[t], recv_sem=sems[1].at[t],
                device_id=(frm,), device_id_type=pl.DeviceIdType.MESH,
            )

        in_flight = []
        for t in (0, 1):
            for rightward in (True, False):
                d = send_desc(t, rightward)
                d.start()
                in_flight.append(d)
        for t in range(2, SUB_R):
            recv_desc(t - 2, True).wait_recv()
            d = send_desc(t, True)
            d.start()
            in_flight.append(d)
            if t < SUB_L:
                recv_desc(t - 2, False).wait_recv()
                d = send_desc(t, False)
                d.start()
                in_flight.append(d)
        for t in (SUB_R - 2, SUB_R - 1):
            recv_desc(t, True).wait_recv()
        for t in (SUB_L - 2, SUB_L - 1):
            recv_desc(t, False).wait_recv()
        for d in in_flight:
            d.wait_send()
        own.wait()

    pos_tab = jnp.asarray(POS, dtype=jnp.int32)
    ring_tab = jnp.asarray(RING, dtype=jnp.int32)

    return pl.pallas_call(
        body,
        out_shape=jax.ShapeDtypeStruct((N_DEV * m_per, n), x.dtype),
        in_specs=[
            pl.BlockSpec(memory_space=pltpu.VMEM),
            pl.BlockSpec(memory_space=pltpu.SMEM),
            pl.BlockSpec(memory_space=pltpu.SMEM),
        ],
        out_specs=pl.BlockSpec(memory_space=pl.ANY),
        scratch_shapes=[
            pltpu.SemaphoreType.DMA((SUB_R,)),
            pltpu.SemaphoreType.DMA((SUB_R,)),
            pltpu.SemaphoreType.DMA((SUB_L,)),
            pltpu.SemaphoreType.DMA((SUB_L,)),
            pltpu.SemaphoreType.DMA,
        ],
        compiler_params=pltpu.CompilerParams(collective_id=0),
    )(x, pos_tab, ring_tab)
